# baseline (speedup 1.0000x reference)
import jax
import jax.numpy as jnp
from jax import lax
from jax.experimental import pallas as pl
from jax.experimental.pallas import tpu as pltpu
from jax.experimental.pallas import tpu_sc as plsc

V = 1000000
D = 32
B = 4096
H = 200
NC = 2
NS = 16
NW = NC * NS
BPT = B // NW
IPT = BPT * H
C0 = 128
C1 = H - C0
L = 16
NBUF = 4
GRP = BPT // NBUF

TR_COLS = 16384
TR_GRID = (V + TR_COLS - 1) // TR_COLS
VP = TR_GRID * TR_COLS
TR_Q = TR_COLS // 4


def _tr_body(tt_ref, out_ref):
    q = TR_Q
    c = tt_ref[...].astype(jnp.bfloat16).T
    out_ref[...] = jnp.concatenate(
        [c[q * a:q * (a + 1), :] for a in range(4)], axis=1)


def _copies(table_hbm, idx_v, rows_v, sem, b, j):
    off0 = pl.multiple_of(b * H, 8)
    off1 = pl.multiple_of(b * H + C0, 8)
    c0 = pltpu.make_async_copy(table_hbm.at[idx_v.at[pl.ds(off0, C0)]],
                               rows_v.at[j].at[pl.ds(0, C0)], sem)
    c1 = pltpu.make_async_copy(table_hbm.at[idx_v.at[pl.ds(off1, C1)]],
                               rows_v.at[j].at[pl.ds(C0, C1)], sem)
    return c0, c1


def _body(table_hbm, idx_hbm, len_hbm, out_hbm, idx_v, len_v, rows_v, out_v,
          s0, s1, s2, s3):
    sems = (s0, s1, s2, s3)
    wid = lax.axis_index("s") * NC + lax.axis_index("c")
    base_b = pl.multiple_of(wid * BPT, 8)
    base_i = pl.multiple_of(wid * IPT, 8)
    pltpu.sync_copy(idx_hbm.at[pl.ds(base_i, IPT)], idx_v)
    pltpu.sync_copy(len_hbm.at[pl.ds(base_b, BPT)], len_v)

    def start(b, j):
        c0, c1 = _copies(table_hbm, idx_v, rows_v, sems[j], b, j)
        c0.start()
        c1.start()

    def wait(b, j):
        c0, c1 = _copies(table_hbm, idx_v, rows_v, sems[j], b, j)
        c0.wait()
        c1.wait()

    for j in range(NBUF):
        start(j, j)

    def group_body(g, carry):
        for j in range(NBUF):
            b = g * NBUF + j
            wait(b, j)

            def red(i, accs):
                a0, a1 = accs
                for u in range(8):
                    r = i * 8 + u
                    xi = rows_v[j, r, :]
                    a0 = a0 + jax.lax.bitcast_convert_type(
                        xi << 16, jnp.float32)
                    a1 = a1 + jax.lax.bitcast_convert_type(
                        xi & jnp.int32(-65536), jnp.float32)
                return a0, a1

            z = jnp.zeros((L,), jnp.float32)
            a0, a1 = lax.fori_loop(0, H // 8, red, (z, z))
            out_v[b, pl.ds(0, L)] = a0 / len_v[b, pl.ds(0, L)]
            out_v[b, pl.ds(L, L)] = a1 / len_v[b, pl.ds(L, L)]
            start(jnp.minimum(b + NBUF, BPT - 1), j)
        return carry

    lax.fori_loop(0, GRP, group_body, 0)
    for j in range(NBUF):
        wait(BPT - 1, j)
    pltpu.sync_copy(out_v, out_hbm.at[pl.ds(base_b, BPT)])


@jax.jit
def _pooled(table, idx_flat, len2d):
    tt = table.T
    t4 = pl.pallas_call(
        _tr_body,
        grid=(TR_GRID,),
        in_specs=[pl.BlockSpec((D, TR_COLS), lambda i: (0, i))],
        out_specs=pl.BlockSpec((TR_COLS // 4, 128), lambda i: (i, 0)),
        out_shape=jax.ShapeDtypeStruct((VP * D // 128, 128), jnp.bfloat16),
    )(tt)
    pairs = t4.reshape(-1).reshape(VP * D // 2, 2)
    table_lin = jax.lax.bitcast_convert_type(pairs, jnp.int32).reshape(VP, D // 2)

    mesh = plsc.VectorSubcoreMesh(core_axis_name="c", subcore_axis_name="s")
    k = pl.kernel(
        _body,
        mesh=mesh,
        out_type=jax.ShapeDtypeStruct((B, D), jnp.float32),
        scratch_types=[
            pltpu.VMEM((IPT,), jnp.int32),
            pltpu.VMEM((BPT, D), jnp.float32),
            pltpu.VMEM((NBUF, H, D // 2), jnp.int32),
            pltpu.VMEM((BPT, D), jnp.float32),
            pltpu.SemaphoreType.DMA,
            pltpu.SemaphoreType.DMA,
            pltpu.SemaphoreType.DMA,
            pltpu.SemaphoreType.DMA,
        ],
        compiler_params=pltpu.CompilerParams(use_tc_tiling_on_sc=False),
    )
    return k(table_lin, idx_flat, len2d)


def kernel(glove_embeddings, indices, lengths):
    v = indices.astype(jnp.int32).reshape(-1)
    idx_flat = (v & -TR_COLS) | ((v & (TR_Q - 1)) << 2) | ((v // TR_Q) & 3)
    dmap = jnp.concatenate([2 * jnp.arange(16), 2 * jnp.arange(16) + 1])
    perm = jnp.argsort(dmap)
    len2d = lengths.reshape(B, D)[:, dmap]
    out = _pooled(glove_embeddings, idx_flat, len2d)
    return out[:, perm].reshape(-1, 1)


# speedup vs baseline: 34.7497x; 34.7497x over previous
import jax
import jax.numpy as jnp
from jax import lax
from jax.experimental import pallas as pl
from jax.experimental.pallas import tpu as pltpu
from jax.experimental.pallas import tpu_sc as plsc

V = 1000000
D = 32
B = 4096
H = 200
NC = 2
NS = 16
NW = NC * NS
BPT = B // NW
IPT = BPT * H
C0 = 128
C1 = H - C0
L = 16
NBUF = 4
GRP = BPT // NBUF

TR_COLS = 16384
TR_GRID = (V + TR_COLS - 1) // TR_COLS
VP = TR_GRID * TR_COLS
TR_Q = TR_COLS // 4


def _tr_body(tt_ref, out_ref):
    q = TR_Q
    c = tt_ref[...].T
    out_ref[...] = jnp.concatenate(
        [c[q * a:q * (a + 1), :] for a in range(4)], axis=1)


def _copies(table_hbm, idx_v, rows_v, sem, b, j):
    off0 = pl.multiple_of(b * H, 8)
    off1 = pl.multiple_of(b * H + C0, 8)
    c0 = pltpu.make_async_copy(table_hbm.at[idx_v.at[pl.ds(off0, C0)]],
                               rows_v.at[j].at[pl.ds(0, C0)], sem)
    c1 = pltpu.make_async_copy(table_hbm.at[idx_v.at[pl.ds(off1, C1)]],
                               rows_v.at[j].at[pl.ds(C0, C1)], sem)
    return c0, c1


def _body(table_hbm, idx_hbm, len_hbm, out_hbm, idx_v, len_v, rows_v, out_v,
          s0, s1, s2, s3):
    sems = (s0, s1, s2, s3)
    wid = lax.axis_index("s") * NC + lax.axis_index("c")
    base_b = pl.multiple_of(wid * BPT, 8)
    base_i = pl.multiple_of(wid * IPT, 8)
    pltpu.sync_copy(idx_hbm.at[pl.ds(base_i, IPT)], idx_v)
    pltpu.sync_copy(len_hbm.at[pl.ds(base_b, BPT)], len_v)

    def start(b, j):
        c0, c1 = _copies(table_hbm, idx_v, rows_v, sems[j], b, j)
        c0.start()
        c1.start()

    def wait(b, j):
        c0, c1 = _copies(table_hbm, idx_v, rows_v, sems[j], b, j)
        c0.wait()
        c1.wait()

    for j in range(NBUF):
        start(j, j)

    def group_body(g, carry):
        for j in range(NBUF):
            b = g * NBUF + j
            wait(b, j)

            def red(i, accs):
                a0, a1 = accs
                for u in range(8):
                    r = i * 8 + u
                    a0 = a0 + rows_v[j, r, pl.ds(0, L)]
                    a1 = a1 + rows_v[j, r, pl.ds(L, L)]
                return a0, a1

            z = jnp.zeros((L,), jnp.float32)
            a0, a1 = lax.fori_loop(0, H // 8, red, (z, z))
            out_v[b, pl.ds(0, L)] = a0 / len_v[b, pl.ds(0, L)]
            out_v[b, pl.ds(L, L)] = a1 / len_v[b, pl.ds(L, L)]
            start(jnp.minimum(b + NBUF, BPT - 1), j)
        return carry

    lax.fori_loop(0, GRP, group_body, 0)
    for j in range(NBUF):
        wait(BPT - 1, j)
    pltpu.sync_copy(out_v, out_hbm.at[pl.ds(base_b, BPT)])


@jax.jit
def _pooled(table, idx_flat, len2d):
    tt = table.T
    t4 = pl.pallas_call(
        _tr_body,
        grid=(TR_GRID,),
        in_specs=[pl.BlockSpec((D, TR_COLS), lambda i: (0, i))],
        out_specs=pl.BlockSpec((TR_COLS // 4, 128), lambda i: (i, 0)),
        out_shape=jax.ShapeDtypeStruct((VP * D // 128, 128), jnp.float32),
    )(tt)
    table_lin = t4.reshape(-1).reshape(VP, D)

    mesh = plsc.VectorSubcoreMesh(core_axis_name="c", subcore_axis_name="s")
    k = pl.kernel(
        _body,
        mesh=mesh,
        out_type=jax.ShapeDtypeStruct((B, D), jnp.float32),
        scratch_types=[
            pltpu.VMEM((IPT,), jnp.int32),
            pltpu.VMEM((BPT, D), jnp.float32),
            pltpu.VMEM((NBUF, H, D), jnp.float32),
            pltpu.VMEM((BPT, D), jnp.float32),
            pltpu.SemaphoreType.DMA,
            pltpu.SemaphoreType.DMA,
            pltpu.SemaphoreType.DMA,
            pltpu.SemaphoreType.DMA,
        ],
        compiler_params=pltpu.CompilerParams(use_tc_tiling_on_sc=False),
    )
    return k(table_lin, idx_flat, len2d)


def kernel(glove_embeddings, indices, lengths):
    v = indices.astype(jnp.int32).reshape(-1)
    idx_flat = (v & -TR_COLS) | ((v & (TR_Q - 1)) << 2) | ((v // TR_Q) & 3)
    len2d = lengths.reshape(B, D)
    out = _pooled(glove_embeddings, idx_flat, len2d)
    return out.reshape(-1, 1)
